# baseline (speedup 1.0000x reference)
import functools

import jax
import jax.numpy as jnp
from jax import lax
from jax.experimental import pallas as pl
from jax.experimental.pallas import tpu as pltpu
from jax.experimental.pallas import tpu_sc as plsc

NUM_CORES = 2
NUM_SUBCORES = 16
LANES = 16
NUM_WORKERS = NUM_CORES * NUM_SUBCORES

BATCH = 16384
DIM = 32
BPW = BATCH // NUM_WORKERS
CHUNK = 128
NCHUNK = BPW // CHUNK
GROUPS = BPW // LANES


def _score_body(h_hbm, r_hbm, t_hbm, ent_hbm, rel_hbm, out_hbm,
                h_v, r_v, t_v, he_v, re_v, te_v, out_v,
                sem_h, sem_r, sem_t):
    wid = lax.axis_index("s") * NUM_CORES + lax.axis_index("c")
    base = wid * NCHUNK
    pltpu.sync_copy(h_hbm.at[pl.ds(base, NCHUNK)], h_v)
    pltpu.sync_copy(r_hbm.at[pl.ds(base, NCHUNK)], r_v)
    pltpu.sync_copy(t_hbm.at[pl.ds(base, NCHUNK)], t_v)

    copies = []
    for c in range(NCHUNK):
        dst = pl.ds(c * CHUNK, CHUNK)
        copies.append(pltpu.async_copy(ent_hbm.at[h_v.at[c]], he_v.at[dst], sem_h))
        copies.append(pltpu.async_copy(rel_hbm.at[r_v.at[c]], re_v.at[dst], sem_r))
        copies.append(pltpu.async_copy(ent_hbm.at[t_v.at[c]], te_v.at[dst], sem_t))
    for cp in copies:
        cp.wait()

    iota = lax.iota(jnp.int32, LANES)

    def group(g, carry):
        row = iota + g * LANES
        acc = jnp.zeros((LANES,), jnp.float32)
        for j in range(DIM):
            col = lax.rem(iota + j, DIM)
            he = plsc.load_gather(he_v, [row, col])
            re = plsc.load_gather(re_v, [row, col])
            te = plsc.load_gather(te_v, [row, col])
            d = he + re - te
            acc = acc + d * d
        x = acc + 1e-12
        i = plsc.bitcast(x, jnp.int32)
        i = jnp.int32(0x5F3759DF) - (i >> 1)
        y = plsc.bitcast(i, jnp.float32)
        for _ in range(3):
            y = y * (1.5 - 0.5 * x * y * y)
        out_v[pl.ds(pl.multiple_of(g * LANES, LANES), LANES)] = -(x * y)
        return carry

    lax.fori_loop(0, GROUPS, group, 0)
    pltpu.sync_copy(out_v, out_hbm.at[pl.ds(wid * BPW, BPW)])


def kernel(h, r, t, ent_emb, rel_emb):
    h2 = h.astype(jnp.int32).reshape(NUM_WORKERS * NCHUNK, CHUNK)
    r2 = r.astype(jnp.int32).reshape(NUM_WORKERS * NCHUNK, CHUNK)
    t2 = t.astype(jnp.int32).reshape(NUM_WORKERS * NCHUNK, CHUNK)
    mesh = plsc.VectorSubcoreMesh(core_axis_name="c", subcore_axis_name="s")
    fn = pl.kernel(
        _score_body,
        mesh=mesh,
        compiler_params=pltpu.CompilerParams(
            needs_layout_passes=False, use_tc_tiling_on_sc=False
        ),
        out_type=jax.ShapeDtypeStruct((BATCH,), jnp.float32),
        scratch_types=[
            pltpu.VMEM((NCHUNK, CHUNK), jnp.int32),
            pltpu.VMEM((NCHUNK, CHUNK), jnp.int32),
            pltpu.VMEM((NCHUNK, CHUNK), jnp.int32),
            pltpu.VMEM((BPW, DIM), jnp.float32),
            pltpu.VMEM((BPW, DIM), jnp.float32),
            pltpu.VMEM((BPW, DIM), jnp.float32),
            pltpu.VMEM((BPW,), jnp.float32),
            pltpu.SemaphoreType.DMA,
            pltpu.SemaphoreType.DMA,
            pltpu.SemaphoreType.DMA,
        ],
    )
    return fn(h2, r2, t2, ent_emb, rel_emb)
